# baseline (speedup 1.0000x reference)
import functools

import jax
import jax.numpy as jnp
from jax import lax
from jax.experimental import pallas as pl
from jax.experimental.pallas import tpu as pltpu
from jax.experimental.pallas import tpu_sc as plsc

_BATCH = 4096
_EMBED = 64
_LANES = 16
_NROWS = 100000
_NBLK = (_NROWS + 127) // 128
_CBLK = 4
_NCH = 7
_RING = 12
_OUTPAD = 128
_TB = (_NBLK - _CBLK) * 128


def _iota16():
    return lax.iota(jnp.int32, _LANES)


@functools.lru_cache(maxsize=None)
def _build():
    info = plsc.get_sparse_core_info()
    nc = info.num_cores
    nw = info.num_cores * info.num_subcores

    mesh = plsc.VectorSubcoreMesh(core_axis_name="c", subcore_axis_name="s")

    @functools.partial(
        pl.kernel,
        mesh=mesh,
        compiler_params=pltpu.CompilerParams(
            needs_layout_passes=False,
            skip_device_barrier=True,
        ),
        out_type=(
            jax.ShapeDtypeStruct((_BATCH + nw, _OUTPAD), jnp.float32),
            jax.ShapeDtypeStruct((_BATCH + nw, _OUTPAD), jnp.float32),
        ),
        scratch_types=[
            pltpu.VMEM((_BATCH,), jnp.int32),
            pltpu.VMEM((_BATCH,), jnp.int32),
            pltpu.VMEM((_BATCH + _LANES,), jnp.int32),
            pltpu.VMEM((_BATCH + _LANES,), jnp.int32),
            pltpu.VMEM((_BATCH + _LANES,), jnp.int32),
            pltpu.VMEM((_BATCH + _LANES,), jnp.int32),
            pltpu.VMEM((_BATCH + _LANES,), jnp.int32),
            pltpu.VMEM((_BATCH + _LANES,), jnp.int32),
            pltpu.VMEM((_EMBED, 128 * _CBLK), jnp.float32),
            pltpu.VMEM((_EMBED, 128 * _CBLK), jnp.float32),
            pltpu.VMEM((_RING * _LANES, _OUTPAD), jnp.float32),
            pltpu.SemaphoreType.DMA,
            pltpu.SemaphoreType.DMA,
            pltpu.SemaphoreType.DMA,
        ],
    )
    def gather_kernel(user_hbm, item_hbm, utab_hbm, itab_hbm,
                      utail_hbm, itail_hbm, uout_hbm, iout_hbm,
                      uidx_v, iidx_v, hidx_v, hpos_v, hidx2_v, hpos2_v,
                      bidx_v, bpos_v,
                      slab_a, slab_b, ring_v, sem_a, sem_b, sem_s):
        wid = lax.axis_index("s") * nc + lax.axis_index("c")
        slabs = (slab_a, slab_b)
        sems = (sem_a, sem_b)

        pltpu.sync_copy(user_hbm, uidx_v)
        pltpu.sync_copy(item_hbm, iidx_v)

        lo = jnp.where(wid < 14, 25 * wid, 24 * wid + 14)
        hi_w = lo + jnp.where(wid < 14, 25, 24)

        def scan_both():
            @pl.loop(0, _BATCH // _LANES, step=2,
                     init_carry=(jnp.int32(0), jnp.int32(0)))
            def cnts(k0, cnts):
                ucnt, icnt = cnts
                vs, ms, cs = [], [], []
                for arr in (uidx_v, iidx_v):
                    for u in range(2):
                        v = arr[pl.ds((k0 + u) * _LANES, _LANES)]
                        ell = v >> 7
                        w1 = (ell * 2622) >> 16
                        w2 = 14 + (((ell - 350) * 2731) >> 16)
                        w_of = jnp.where(ell < 350, w1, w2)
                        m = w_of == wid
                        vs.append(v)
                        ms.append(m)
                        cs.append(jnp.sum(m.astype(jnp.int32)))
                for u in range(2):
                    plsc.store_compressed(
                        hidx_v.at[pl.ds(ucnt, _LANES)], vs[u], mask=ms[u])
                    plsc.store_compressed(
                        hpos_v.at[pl.ds(ucnt, _LANES)],
                        _iota16() + (k0 + u) * _LANES, mask=ms[u])
                    ucnt = ucnt + cs[u]
                for u in range(2):
                    plsc.store_compressed(
                        hidx2_v.at[pl.ds(icnt, _LANES)], vs[2 + u],
                        mask=ms[2 + u])
                    plsc.store_compressed(
                        hpos2_v.at[pl.ds(icnt, _LANES)],
                        _iota16() + (k0 + u) * _LANES, mask=ms[2 + u])
                    icnt = icnt + cs[2 + u]
                return ucnt, icnt
            return cnts

        def chunk_base(c):
            cl = lo + _CBLK * c
            return pl.multiple_of(cl * 128, 128), cl

        def chunk_copy(tab, tail_tab, c, par):
            base, cl = chunk_base(c)
            full = pltpu.make_async_copy(
                tab.at[:, pl.ds(base, 128 * _CBLK)], slabs[par], sems[par])
            tail = pltpu.make_async_copy(tail_tab, slabs[par], sems[par])
            return cl, base, full, tail

        def start_chunk(tab, tail_tab, c, par):
            cl, base, full, tail = chunk_copy(tab, tail_tab, c, par)

            @pl.when((cl < hi_w) & (base < _TB))
            def _():
                full.start()

            @pl.when((cl < hi_w) & (base == _TB))
            def _():
                tail.start()

        def wait_chunk(tab, tail_tab, c, par):
            cl, base, full, tail = chunk_copy(tab, tail_tab, c, par)

            @pl.when((cl < hi_w) & (base < _TB))
            def _():
                full.wait()

            @pl.when((cl < hi_w) & (base == _TB))
            def _():
                tail.wait()

        def process_table(tab, tail_tab, out_hbm, hit_idx, hit_pos, cnt,
                          gcnt, primed):
            cnt_spl = jnp.full((_LANES,), cnt, jnp.int32)
            nhv = (cnt + _LANES - 1) // _LANES

            if not primed:
                start_chunk(tab, tail_tab, 0, 0)

            @pl.loop(0, _NCH + 1, step=2, init_carry=gcnt)
            def gcnt(c0, gcnt):
                for b in range(2):
                    c = c0 + b
                    start_chunk(tab, tail_tab, c + 1, (b + 1) % 2)
                    base, cl = chunk_base(c)
                    cl_spl = jnp.full((_LANES,), cl, jnp.int32)
                    cu_spl = jnp.full(
                        (_LANES,), jnp.minimum(cl + _CBLK, hi_w), jnp.int32)
                    base_spl = jnp.full((_LANES,), base, jnp.int32)

                    @pl.loop(0, nhv, init_carry=jnp.int32(0))
                    def bcnt(hv, bcnt):
                        hi = hit_idx[pl.ds(hv * _LANES, _LANES)]
                        hp = hit_pos[pl.ds(hv * _LANES, _LANES)]
                        valid = (_iota16() + hv * _LANES) < cnt_spl
                        ell = hi >> 7
                        m = (ell >= cl_spl) & (ell < cu_spl) & valid
                        plsc.store_compressed(
                            bidx_v.at[pl.ds(bcnt, _LANES)], hi, mask=m)
                        plsc.store_compressed(
                            bpos_v.at[pl.ds(bcnt, _LANES)], hp, mask=m)
                        return bcnt + jnp.sum(m.astype(jnp.int32))

                    wait_chunk(tab, tail_tab, c, b)
                    slab = slabs[b]
                    bcnt_spl = jnp.full((_LANES,), bcnt, jnp.int32)
                    nbv = (bcnt + _LANES - 1) // _LANES

                    @pl.loop(0, nbv, init_carry=gcnt)
                    def gcnt(hv, gcnt):
                        lanevec = (bidx_v[pl.ds(hv * _LANES, _LANES)]
                                   - base_spl) & (128 * _CBLK - 1)
                        pv = bpos_v[pl.ds(hv * _LANES, _LANES)]
                        valid = (_iota16() + hv * _LANES) < bcnt_spl
                        pv = jnp.where(valid, pv, _BATCH + wid)

                        @pl.when(gcnt >= _RING)
                        def _():
                            pltpu.make_async_copy(
                                out_hbm.at[pl.ds(0, _LANES)],
                                ring_v.at[pl.ds(0, _LANES)], sem_s).wait()

                        slot = (gcnt % _RING) * _LANES
                        rows = _iota16() + slot
                        for e in range(_EMBED):
                            v = plsc.load_gather(
                                slab,
                                [jnp.full((_LANES,), e, jnp.int32),
                                 lanevec])
                            plsc.store_scatter(
                                ring_v,
                                [rows, jnp.full((_LANES,), e, jnp.int32)],
                                v)
                        pltpu.async_copy(
                            ring_v.at[pl.ds(slot, _LANES)],
                            out_hbm.at[pv], sem_s)
                        return gcnt + 1

                return gcnt

            return gcnt

        start_chunk(utab_hbm, utail_hbm, 0, 0)
        ucnt, icnt = scan_both()
        gcnt = process_table(utab_hbm, utail_hbm, uout_hbm, hidx_v, hpos_v,
                             ucnt, jnp.int32(0), True)
        gcnt = process_table(itab_hbm, itail_hbm, iout_hbm, hidx2_v,
                             hpos2_v, icnt, gcnt, False)

        @pl.loop(0, jnp.minimum(gcnt, _RING))
        def _(i):
            pltpu.make_async_copy(
                uout_hbm.at[pl.ds(0, _LANES)],
                ring_v.at[pl.ds(0, _LANES)], sem_s).wait()

    return gather_kernel


def kernel(user, item, user_table, item_table):
    gather_kernel = _build()
    ut3 = jnp.transpose(user_table)
    it3 = jnp.transpose(item_table)
    pad = 128 * _CBLK - (_NROWS - _TB)
    utail = jnp.pad(jnp.transpose(user_table[_TB:, :]), ((0, 0), (0, pad)))
    itail = jnp.pad(jnp.transpose(item_table[_TB:, :]), ((0, 0), (0, pad)))
    up, ip = gather_kernel(
        user.astype(jnp.int32), item.astype(jnp.int32), ut3, it3,
        utail, itail)
    return (up[:_BATCH, :_EMBED], ip[:_BATCH, :_EMBED])
